# baseline (speedup 1.0000x reference)
import functools

import jax
import jax.numpy as jnp
from jax.experimental import pallas as pl
from jax.experimental.pallas import tpu as pltpu

_ALPHA = (0.12, 0.31, 0.44, 0.27, 0.53, 0.19, 0.66, 0.38,
          0.22, 0.49, 0.17, 0.61, 0.34, 0.28, 0.57, 0.41)
_GAMMA = 2.0

_TR = 8
_GT = 4
_HEAD = 6
_DEPTH = 3


def _tree(fn, xs):
    xs = list(xs)
    while len(xs) > 1:
        nxt = [fn(xs[i], xs[i + 1]) for i in range(0, len(xs) - 1, 2)]
        if len(xs) % 2:
            nxt.append(xs[-1])
        xs = nxt
    return xs[0]


def _bit_select(vals, t, nbits):
    cur = list(vals)
    for k in range(nbits):
        bit = (t & (1 << k)) != 0
        cur = [jnp.where(bit, cur[2 * i + 1], cur[2 * i])
               for i in range(len(cur) // 2)]
    return cur[0]


def _tile_loss(xbuf, tbuf, s, C, rows, alpha, nbits, gt=_GT):
    acc = None
    for g in range(gt):
        for r0 in range(0, rows, _TR):
            sl = slice(r0, r0 + _TR)
            xs = [xbuf[s, g, c, sl, :] for c in range(C)]
            t = tbuf[s, g, 0, sl, :]

            m = _tree(jnp.maximum, xs)
            ds = [x - m for x in xs]
            se = _tree(lambda a, b: a + b, [jnp.exp(d) for d in ds])

            dt = _bit_select(ds, t, nbits)
            at = _bit_select([jnp.float32(a) for a in alpha], t, nbits)

            logpt = dt - jnp.log(se)
            pt = jnp.exp(logpt)
            omp = 1.0 - pt
            contrib = (omp * omp) * (at * logpt)
            acc = contrib if acc is None else acc + contrib
    return acc


def _focal_kernel(x_hbm, t_hbm, out_ref, xbuf, tbuf, xsem, tsem, *, C, rows,
                  mid, edge, alpha, nbits, inv_m):
    def start(off, n, s):
        pltpu.make_async_copy(
            x_hbm.at[pl.ds(off, n)], xbuf.at[s, pl.ds(0, n)], xsem.at[s]).start()
        pltpu.make_async_copy(
            t_hbm.at[pl.ds(off, n)], tbuf.at[s, pl.ds(0, n)], tsem.at[s]).start()

    def wait(n, s):
        pltpu.make_async_copy(
            xbuf.at[s, pl.ds(0, n)], xbuf.at[s, pl.ds(0, n)], xsem.at[s]).wait()
        pltpu.make_async_copy(
            tbuf.at[s, pl.ds(0, n)], tbuf.at[s, pl.ds(0, n)], tsem.at[s]).wait()

    head = _HEAD
    assert mid >= 1
    off = lambda j: head + (j - 1) * _GT
    tail_off = head + mid * _GT
    last_slot = (mid + 1) % _DEPTH

    start(0, head, 0)
    start(off(1), _GT, 1)
    if mid >= 2:
        start(off(2), _GT, 2)
    else:
        start(tail_off, edge, 2)

    wait(head, 0)
    total = _tile_loss(xbuf, tbuf, 0, C, rows, alpha, nbits, gt=head)
    if _DEPTH <= mid:
        start(off(_DEPTH), _GT, 0)
    elif _DEPTH == mid + 1:
        start(tail_off, edge, 0)

    def body(k, total):
        s = jax.lax.rem(k + 1, _DEPTH)
        wait(_GT, s)
        acc = _tile_loss(xbuf, tbuf, s, C, rows, alpha, nbits)

        @pl.when(k + _DEPTH + 1 <= mid)
        def _():
            start(head + (k + _DEPTH) * _GT, _GT, s)

        @pl.when(k + _DEPTH == mid)
        def _():
            start(tail_off, edge, s)

        return total + acc

    total = jax.lax.fori_loop(0, mid, body, total)

    wait(edge, last_slot)
    total = total + _tile_loss(xbuf, tbuf, last_slot, C, rows, alpha, nbits,
                               gt=edge)
    out_ref[0, 0] = jnp.sum(total) * jnp.float32(-inv_m)


def kernel(logits, target):
    N, C = logits.shape[0], logits.shape[1]
    HW = 1
    for d in logits.shape[2:]:
        HW *= d
    M = N * HW
    assert HW % 128 == 0, "token count must be lane aligned"
    R = HW // 128

    x = logits.reshape(N, C, R, 128)
    t = target.reshape(N, 1, R, 128)

    edge = 2
    assert (N - _HEAD - edge) % _GT == 0, "image count must tile evenly"
    mid = (N - _HEAD - edge) // _GT
    nbits = max(1, (C - 1).bit_length())
    assert C == len(_ALPHA) and (1 << nbits) == C
    assert R % _TR == 0

    kern = functools.partial(_focal_kernel, C=C, rows=R, mid=mid, edge=edge,
                             alpha=_ALPHA, nbits=nbits, inv_m=1.0 / M)
    total = pl.pallas_call(
        kern,
        out_shape=jax.ShapeDtypeStruct((1, 1), jnp.float32),
        in_specs=[
            pl.BlockSpec(memory_space=pl.ANY),
            pl.BlockSpec(memory_space=pl.ANY),
        ],
        out_specs=pl.BlockSpec(memory_space=pltpu.SMEM),
        scratch_shapes=[
            pltpu.VMEM((_DEPTH, max(_HEAD, _GT), C, R, 128), jnp.float32),
            pltpu.VMEM((_DEPTH, max(_HEAD, _GT), 1, R, 128), jnp.int32),
            pltpu.SemaphoreType.DMA((_DEPTH,)),
            pltpu.SemaphoreType.DMA((_DEPTH,)),
        ],
        compiler_params=pltpu.CompilerParams(
            vmem_limit_bytes=48 * 1024 * 1024),
    )(x, t)
    return total.reshape(())


# speedup vs baseline: 1.0571x; 1.0571x over previous
import functools

import jax
import jax.numpy as jnp
from jax.experimental import pallas as pl
from jax.experimental.pallas import tpu as pltpu

_ALPHA = (0.12, 0.31, 0.44, 0.27, 0.53, 0.19, 0.66, 0.38,
          0.22, 0.49, 0.17, 0.61, 0.34, 0.28, 0.57, 0.41)
_GAMMA = 2.0

_TR = 8
_GT = 4
_DEPTH = 3


def _tree(fn, xs):
    xs = list(xs)
    while len(xs) > 1:
        nxt = [fn(xs[i], xs[i + 1]) for i in range(0, len(xs) - 1, 2)]
        if len(xs) % 2:
            nxt.append(xs[-1])
        xs = nxt
    return xs[0]


def _bit_select(vals, t, nbits):
    cur = list(vals)
    for k in range(nbits):
        bit = (t & (1 << k)) != 0
        cur = [jnp.where(bit, cur[2 * i + 1], cur[2 * i])
               for i in range(len(cur) // 2)]
    return cur[0]


def _tile_loss(xbuf, tbuf, s, C, rows, alpha, nbits):
    acc = None
    for g in range(_GT):
        for r0 in range(0, rows, _TR):
            sl = slice(r0, r0 + _TR)
            xs = [xbuf[s, g, c, sl, :] for c in range(C)]
            t = tbuf[s, g, 0, sl, :]

            m = _tree(jnp.maximum, xs)
            ds = [x - m for x in xs]
            se = _tree(lambda a, b: a + b, [jnp.exp(d) for d in ds])

            dt = _bit_select(ds, t, nbits)
            at = _bit_select([jnp.float32(a) for a in alpha], t, nbits)

            logpt = dt - jnp.log(se)
            pt = jnp.exp(logpt)
            omp = 1.0 - pt
            contrib = (omp * omp) * (at * logpt)
            acc = contrib if acc is None else acc + contrib
    return acc


def _focal_kernel(x_hbm, t_hbm, out_ref, xbuf, tbuf, xsem, tsem, *, C, rows,
                  tiles, alpha, nbits, inv_m):
    def start(k, s):
        pltpu.make_async_copy(
            x_hbm.at[pl.ds(k * _GT, _GT)], xbuf.at[s], xsem.at[s]).start()
        pltpu.make_async_copy(
            t_hbm.at[pl.ds(k * _GT, _GT)], tbuf.at[s], tsem.at[s]).start()

    for k in range(min(_DEPTH, tiles)):
        start(k, k % _DEPTH)

    def body(k, total):
        s = jax.lax.rem(k, _DEPTH)
        pltpu.make_async_copy(xbuf.at[s], xbuf.at[s], xsem.at[s]).wait()
        pltpu.make_async_copy(tbuf.at[s], tbuf.at[s], tsem.at[s]).wait()
        acc = _tile_loss(xbuf, tbuf, s, C, rows, alpha, nbits)

        @pl.when(k + _DEPTH < tiles)
        def _():
            start(k + _DEPTH, s)

        return total + acc

    total = jax.lax.fori_loop(0, tiles, body, jnp.zeros((_TR, 128), jnp.float32))
    out_ref[0, 0] = jnp.sum(total) * jnp.float32(-inv_m)


def kernel(logits, target):
    N, C = logits.shape[0], logits.shape[1]
    HW = 1
    for d in logits.shape[2:]:
        HW *= d
    M = N * HW
    assert HW % 128 == 0, "token count must be lane aligned"
    R = HW // 128

    x = logits.reshape(N, C, R, 128)
    t = target.reshape(N, 1, R, 128)

    assert N % _GT == 0, "image count must tile evenly"
    tiles = N // _GT
    nbits = max(1, (C - 1).bit_length())
    assert C == len(_ALPHA) and (1 << nbits) == C
    assert R % _TR == 0

    kern = functools.partial(_focal_kernel, C=C, rows=R, tiles=tiles,
                             alpha=_ALPHA, nbits=nbits, inv_m=1.0 / M)
    total = pl.pallas_call(
        kern,
        out_shape=jax.ShapeDtypeStruct((1, 1), jnp.float32),
        in_specs=[
            pl.BlockSpec(memory_space=pl.ANY),
            pl.BlockSpec(memory_space=pl.ANY),
        ],
        out_specs=pl.BlockSpec(memory_space=pltpu.SMEM),
        scratch_shapes=[
            pltpu.VMEM((_DEPTH, _GT, C, R, 128), jnp.float32),
            pltpu.VMEM((_DEPTH, _GT, 1, R, 128), jnp.int32),
            pltpu.SemaphoreType.DMA((_DEPTH,)),
            pltpu.SemaphoreType.DMA((_DEPTH,)),
        ],
        compiler_params=pltpu.CompilerParams(
            vmem_limit_bytes=48 * 1024 * 1024),
    )(x, t)
    return total.reshape(())


# speedup vs baseline: 1.0611x; 1.0039x over previous
import functools

import jax
import jax.numpy as jnp
from jax.experimental import pallas as pl
from jax.experimental.pallas import tpu as pltpu

_ALPHA = (0.12, 0.31, 0.44, 0.27, 0.53, 0.19, 0.66, 0.38,
          0.22, 0.49, 0.17, 0.61, 0.34, 0.28, 0.57, 0.41)
_GAMMA = 2.0

_TR = 8
_GT = 4
_DEPTH = 3


def _tree(fn, xs):
    xs = list(xs)
    while len(xs) > 1:
        nxt = [fn(xs[i], xs[i + 1]) for i in range(0, len(xs) - 1, 2)]
        if len(xs) % 2:
            nxt.append(xs[-1])
        xs = nxt
    return xs[0]


def _bit_select(vals, t, nbits):
    cur = list(vals)
    for k in range(nbits):
        bit = (t & (1 << k)) != 0
        cur = [jnp.where(bit, cur[2 * i + 1], cur[2 * i])
               for i in range(len(cur) // 2)]
    return cur[0]


def _tile_loss(xbuf, tbuf, s, base, C, rows, alpha, nbits):
    acc = None
    for g in range(_GT):
        for r0 in range(0, rows, _TR):
            sl = slice(r0, r0 + _TR)
            xs = [xbuf[s, g, c, sl, :] for c in range(C)]
            t = tbuf[base + g, 0, sl, :]

            m = _tree(jnp.maximum, xs)
            ds = [x - m for x in xs]
            se = _tree(lambda a, b: a + b, [jnp.exp(d) for d in ds])

            dt = _bit_select(ds, t, nbits)
            at = _bit_select([jnp.float32(a) for a in alpha], t, nbits)

            logpt = dt - jnp.log(se)
            pt = jnp.exp(logpt)
            omp = 1.0 - pt
            contrib = (omp * omp) * (at * logpt)
            acc = contrib if acc is None else acc + contrib
    return acc


def _focal_kernel(x_hbm, t_hbm, out_ref, xbuf, tbuf, xsem, tsem, *, C, rows,
                  tiles, alpha, nbits, inv_m):
    def start(k, s):
        pltpu.make_async_copy(
            x_hbm.at[pl.ds(k * _GT, _GT)], xbuf.at[s], xsem.at[s]).start()

    start(0, 0)
    pltpu.make_async_copy(t_hbm, tbuf, tsem).start()
    for k in range(1, min(_DEPTH, tiles)):
        start(k, k % _DEPTH)
    pltpu.make_async_copy(t_hbm, tbuf, tsem).wait()

    def body(k, total):
        s = jax.lax.rem(k, _DEPTH)
        pltpu.make_async_copy(xbuf.at[s], xbuf.at[s], xsem.at[s]).wait()
        acc = _tile_loss(xbuf, tbuf, s, k * _GT, C, rows, alpha, nbits)

        @pl.when(k + _DEPTH < tiles)
        def _():
            start(k + _DEPTH, s)

        return total + acc

    total = jax.lax.fori_loop(0, tiles, body, jnp.zeros((_TR, 128), jnp.float32))
    out_ref[0, 0] = jnp.sum(total) * jnp.float32(-inv_m)


def kernel(logits, target):
    N, C = logits.shape[0], logits.shape[1]
    HW = 1
    for d in logits.shape[2:]:
        HW *= d
    M = N * HW
    assert HW % 128 == 0, "token count must be lane aligned"
    R = HW // 128

    x = logits.reshape(N, C, R, 128)
    t = target.reshape(N, 1, R, 128)

    assert N % _GT == 0, "image count must tile evenly"
    tiles = N // _GT
    nbits = max(1, (C - 1).bit_length())
    assert C == len(_ALPHA) and (1 << nbits) == C
    assert R % _TR == 0

    kern = functools.partial(_focal_kernel, C=C, rows=R, tiles=tiles,
                             alpha=_ALPHA, nbits=nbits, inv_m=1.0 / M)
    total = pl.pallas_call(
        kern,
        out_shape=jax.ShapeDtypeStruct((1, 1), jnp.float32),
        in_specs=[
            pl.BlockSpec(memory_space=pl.ANY),
            pl.BlockSpec(memory_space=pl.ANY),
        ],
        out_specs=pl.BlockSpec(memory_space=pltpu.SMEM),
        scratch_shapes=[
            pltpu.VMEM((_DEPTH, _GT, C, R, 128), jnp.float32),
            pltpu.VMEM((N, 1, R, 128), jnp.int32),
            pltpu.SemaphoreType.DMA((_DEPTH,)),
            pltpu.SemaphoreType.DMA(()),
        ],
        compiler_params=pltpu.CompilerParams(
            vmem_limit_bytes=48 * 1024 * 1024),
    )(x, t)
    return total.reshape(())


# speedup vs baseline: 1.0621x; 1.0009x over previous
import functools

import jax
import jax.numpy as jnp
from jax.experimental import pallas as pl
from jax.experimental.pallas import tpu as pltpu

_ALPHA = (0.12, 0.31, 0.44, 0.27, 0.53, 0.19, 0.66, 0.38,
          0.22, 0.49, 0.17, 0.61, 0.34, 0.28, 0.57, 0.41)
_GAMMA = 2.0

_TR = 8
_GT = 4
_DEPTH = 3


def _tree(fn, xs):
    xs = list(xs)
    while len(xs) > 1:
        nxt = [fn(xs[i], xs[i + 1]) for i in range(0, len(xs) - 1, 2)]
        if len(xs) % 2:
            nxt.append(xs[-1])
        xs = nxt
    return xs[0]


def _bit_select(vals, t, nbits):
    cur = list(vals)
    for k in range(nbits):
        bit = (t & (1 << k)) != 0
        cur = [jnp.where(bit, cur[2 * i + 1], cur[2 * i])
               for i in range(len(cur) // 2)]
    return cur[0]


def _tile_loss(xbuf, tbuf, s, base, C, rows, alpha, nbits):
    acc = None
    for g in range(_GT):
        for r0 in range(0, rows, _TR):
            sl = slice(r0, r0 + _TR)
            xs = [xbuf[s, g, c, sl, :] for c in range(C)]
            t = tbuf[base + g, 0, sl, :]

            m = _tree(jnp.maximum, xs)
            ds = [x - m for x in xs]
            se = _tree(lambda a, b: a + b, [jnp.exp(d) for d in ds])

            dt = _bit_select(ds, t, nbits)
            at = _bit_select([jnp.float32(a) for a in alpha], t, nbits)

            logpt = dt - jnp.log(se)
            pt = jnp.exp(logpt)
            omp = 1.0 - pt
            assert _GAMMA == 2.0
            contrib = (omp * omp) * (at * logpt)
            acc = contrib if acc is None else acc + contrib
    return acc


def _focal_kernel(x_hbm, t_hbm, out_ref, xbuf, tbuf, xsem, tsem, *, C, rows,
                  tiles, alpha, nbits, inv_m):
    def start(k, s):
        pltpu.make_async_copy(
            x_hbm.at[pl.ds(k * _GT, _GT)], xbuf.at[s], xsem.at[s]).start()

    start(0, 0)
    pltpu.make_async_copy(t_hbm, tbuf, tsem).start()
    for k in range(1, min(_DEPTH, tiles)):
        start(k, k % _DEPTH)
    pltpu.make_async_copy(t_hbm, tbuf, tsem).wait()

    def body(k, total):
        s = jax.lax.rem(k, _DEPTH)
        pltpu.make_async_copy(xbuf.at[s], xbuf.at[s], xsem.at[s]).wait()
        acc = _tile_loss(xbuf, tbuf, s, k * _GT, C, rows, alpha, nbits)

        @pl.when(k + _DEPTH < tiles)
        def _():
            start(k + _DEPTH, s)

        return total + acc

    total = jax.lax.fori_loop(0, tiles, body, jnp.zeros((_TR, 128), jnp.float32))
    out_ref[0, 0] = jnp.sum(total) * jnp.float32(-inv_m)


def kernel(logits, target):
    N, C = logits.shape[0], logits.shape[1]
    HW = 1
    for d in logits.shape[2:]:
        HW *= d
    M = N * HW
    assert HW % 128 == 0, "token count must be lane aligned"
    R = HW // 128

    x = logits.reshape(N, C, R, 128)
    t = target.reshape(N, 1, R, 128)

    assert N % _GT == 0, "image count must tile evenly"
    tiles = N // _GT
    nbits = max(1, (C - 1).bit_length())
    assert C == len(_ALPHA) and (1 << nbits) == C
    assert R % _TR == 0

    kern = functools.partial(_focal_kernel, C=C, rows=R, tiles=tiles,
                             alpha=_ALPHA, nbits=nbits, inv_m=1.0 / M)
    total = pl.pallas_call(
        kern,
        out_shape=jax.ShapeDtypeStruct((1, 1), jnp.float32),
        in_specs=[
            pl.BlockSpec(memory_space=pl.ANY),
            pl.BlockSpec(memory_space=pl.ANY),
        ],
        out_specs=pl.BlockSpec(memory_space=pltpu.SMEM),
        scratch_shapes=[
            pltpu.VMEM((_DEPTH, _GT, C, R, 128), jnp.float32),
            pltpu.VMEM((N, 1, R, 128), jnp.int32),
            pltpu.SemaphoreType.DMA((_DEPTH,)),
            pltpu.SemaphoreType.DMA(()),
        ],
        compiler_params=pltpu.CompilerParams(
            vmem_limit_bytes=48 * 1024 * 1024),
    )(x, t)
    return total.reshape(())
